# baseline (speedup 1.0000x reference)
import functools

import jax
import jax.numpy as jnp
from jax import lax
from jax.experimental import pallas as pl
from jax.experimental.pallas import tpu as pltpu
from jax.experimental.pallas import tpu_sc as plsc

N = 10000
E = 320000
D_IN = 128
DH = 64
NC, NS = 2, 16
K = 64
NBUF = 4
PD = 2
NW = NC * NS
_CG = K * NBUF
EPW = ((E + NW - 1) // NW + _CG - 1) // _CG * _CG
NCH = EPW // K
GROUPS = NCH // NBUF
TAIL_E = E - (NW - 1) * EPW
TAILCH = TAIL_E // K
TAILG = TAILCH // NBUF
NPAD = (N // 1280 + 1) * 1280
RPT = NPAD // NS

RB = 1000
NB = N // RB

_mesh = plsc.VectorSubcoreMesh(core_axis_name="c", subcore_axis_name="s")


@functools.partial(
    pl.kernel,
    out_type=[jax.ShapeDtypeStruct((NPAD,), jnp.float32),
              jax.ShapeDtypeStruct((NPAD,), jnp.float32)],
    mesh=_mesh,
    scratch_types=[
        pltpu.VMEM((NCH, K), jnp.int32),
        pltpu.VMEM((K,), jnp.float32),
        pltpu.VMEM((RPT,), jnp.float32),
        pltpu.VMEM_SHARED((NPAD,), jnp.float32),
        pltpu.SemaphoreType.DMA,
    ],
)
def _deg_kernel(dst2_hbm, z_hbm, out0_hbm, out1_hbm, didx2, ones_v, zv, dacc,
                sem):
    c = lax.axis_index("c")
    s = lax.axis_index("s")
    w = c * NS + s
    for j in range(K // 16):
        ones_v[pl.ds(j * 16, 16)] = jnp.full((16,), 1.0, jnp.float32)
    nch_w = jnp.where(w == NW - 1, TAILCH, NCH)
    pltpu.sync_copy(dst2_hbm.at[pl.ds(w * NCH, TAILCH)],
                    didx2.at[pl.ds(0, TAILCH)])
    pl.when(w < NW - 1)(lambda: pltpu.sync_copy(
        dst2_hbm.at[pl.ds(w * NCH + TAILCH, NCH - TAILCH)],
        didx2.at[pl.ds(TAILCH, NCH - TAILCH)]))
    pltpu.sync_copy(z_hbm, zv)
    pltpu.sync_copy(zv, dacc.at[pl.ds(s * RPT, RPT)])
    plsc.subcore_barrier()

    def fire(i, carry):
        pltpu.async_copy(ones_v, dacc.at[didx2.at[i]], sem, add=True)
        return carry

    lax.fori_loop(0, nch_w, fire, 0)

    def drain(i, carry):
        pltpu.make_async_copy(z_hbm.at[pl.ds(0, K)], ones_v, sem).wait()
        return carry

    lax.fori_loop(0, nch_w, drain, 0)
    plsc.subcore_barrier()

    def out(o_hbm):
        pltpu.sync_copy(dacc.at[pl.ds(s * RPT, RPT)],
                        o_hbm.at[pl.ds(s * RPT, RPT)])

    pl.when(c == 0)(lambda: out(out0_hbm))
    pl.when(c == 1)(lambda: out(out1_hbm))


@functools.partial(
    pl.kernel,
    out_type=[jax.ShapeDtypeStruct((NPAD, 128), jnp.float32),
              jax.ShapeDtypeStruct((NPAD, 128), jnp.float32)],
    mesh=_mesh,
    scratch_types=[
        pltpu.VMEM((EPW,), jnp.int32),
        [pltpu.VMEM((K,), jnp.int32)] * NBUF,
        [pltpu.VMEM((K, 128), jnp.float32)] * NBUF,
        pltpu.VMEM_SHARED((NPAD, 128), jnp.float32),
        [pltpu.SemaphoreType.DMA] * NBUF,
        [pltpu.SemaphoreType.DMA] * NBUF,
    ],
)
def _pass_kernel(y_hbm, src1_hbm, dst1_hbm, z_hbm,
                 out0_hbm, out1_hbm, sidx1, didx, rows, acc, gsem, ssem):
    c = lax.axis_index("c")
    s = lax.axis_index("s")
    w = c * NS + s
    gw = jnp.where(w == NW - 1, TAILG, GROUPS)
    pltpu.sync_copy(src1_hbm.at[pl.ds(w * EPW, TAIL_E)],
                    sidx1.at[pl.ds(0, TAIL_E)])
    pl.when(w < NW - 1)(lambda: pltpu.sync_copy(
        src1_hbm.at[pl.ds(w * EPW + TAIL_E, EPW - TAIL_E)],
        sidx1.at[pl.ds(TAIL_E, EPW - TAIL_E)]))
    pltpu.sync_copy(z_hbm, rows[0])
    for t in range(RPT // K):
        pltpu.sync_copy(rows[0], acc.at[pl.ds(s * RPT + t * K, K)])
    plsc.subcore_barrier()

    def launch(m, b):
        pltpu.async_copy(dst1_hbm.at[pl.ds((w * NCH + m) * K, K)], didx[b],
                         gsem[b])
        pltpu.async_copy(y_hbm.at[sidx1.at[pl.ds(m * K, K)]], rows[b],
                         gsem[b])

    def wait_gather(b):
        pltpu.make_async_copy(dst1_hbm.at[pl.ds(0, K)], didx[b],
                              gsem[b]).wait()
        pltpu.make_async_copy(z_hbm, rows[b], gsem[b]).wait()

    def wait_scatter(b):
        pltpu.make_async_copy(z_hbm, rows[b], ssem[b]).wait()

    for t in range(PD):
        launch(t, t)

    def group(g, carry):
        for b in range(NBUF):
            m = g * NBUF + b
            br = (b + PD) % NBUF
            if b + PD < NBUF:
                pl.when(g > 0)(lambda: wait_scatter(br))
                launch(m + PD, br)
            else:
                @pl.when(g < gw - 1)
                def _refill():
                    wait_scatter(br)
                    launch(m + PD, br)

            wait_gather(b)
            pltpu.async_copy(rows[b], acc.at[didx[b]], ssem[b], add=True)
        return carry

    lax.fori_loop(0, gw, group, 0)
    for b in range(NBUF):
        wait_scatter(b)
    plsc.subcore_barrier()

    def out(o_hbm):
        pltpu.sync_copy(acc.at[pl.ds(s * RPT, RPT)],
                        o_hbm.at[pl.ds(s * RPT, RPT)])

    pl.when(c == 0)(lambda: out(out0_hbm))
    pl.when(c == 1)(lambda: out(out1_hbm))


def _tc1_body(x_ref, w_ref, dv_ref, y_ref):
    y = jnp.dot(x_ref[...], w_ref[...], preferred_element_type=jnp.float32)
    y_ref[...] = y * dv_ref[...]


_tc1 = pl.pallas_call(
    _tc1_body,
    grid=(NB,),
    in_specs=[pl.BlockSpec((RB, D_IN), lambda i: (i, 0)),
              pl.BlockSpec((D_IN, 128), lambda i: (0, 0)),
              pl.BlockSpec((RB, 1), lambda i: (i, 0))],
    out_specs=pl.BlockSpec((RB, 128), lambda i: (i, 0)),
    out_shape=jax.ShapeDtypeStruct((N, 128), jnp.float32),
)


def _tc2_body(p0_ref, p1_ref, dv_ref, b1_ref, wm_ref, wl_ref, z_ref):
    dv = dv_ref[...]
    a = p0_ref[...] + p1_ref[...]
    h = jnp.maximum(a * dv + b1_ref[...], 0.0)
    d64 = dv
    zm = jnp.dot(h, wm_ref[...], preferred_element_type=jnp.float32) * d64
    zl = jnp.dot(h, wl_ref[...], preferred_element_type=jnp.float32) * d64
    z_ref[...] = jnp.concatenate([zm, zl], axis=1)


_tc2 = pl.pallas_call(
    _tc2_body,
    grid=(NB,),
    in_specs=[pl.BlockSpec((RB, 128), lambda i: (i, 0)),
              pl.BlockSpec((RB, 128), lambda i: (i, 0)),
              pl.BlockSpec((RB, 1), lambda i: (i, 0)),
              pl.BlockSpec((1, 128), lambda i: (0, 0)),
              pl.BlockSpec((128, DH), lambda i: (0, 0)),
              pl.BlockSpec((128, DH), lambda i: (0, 0))],
    out_specs=pl.BlockSpec((RB, 128), lambda i: (i, 0)),
    out_shape=jax.ShapeDtypeStruct((N, 128), jnp.float32),
)


def _tc3_body(q0_ref, q1_ref, dv_ref, bm_ref, bl_ref, mu_ref, ls_ref):
    d64 = dv_ref[...]
    q = q0_ref[...] + q1_ref[...]
    mu_ref[...] = q[:, :DH] * d64 + bm_ref[...]
    ls_ref[...] = q[:, DH:] * d64 + bl_ref[...]


_tc3 = pl.pallas_call(
    _tc3_body,
    grid=(NB,),
    in_specs=[pl.BlockSpec((RB, 128), lambda i: (i, 0)),
              pl.BlockSpec((RB, 128), lambda i: (i, 0)),
              pl.BlockSpec((RB, 1), lambda i: (i, 0)),
              pl.BlockSpec((1, DH), lambda i: (0, 0)),
              pl.BlockSpec((1, DH), lambda i: (0, 0))],
    out_specs=[pl.BlockSpec((RB, DH), lambda i: (i, 0)),
               pl.BlockSpec((RB, DH), lambda i: (i, 0))],
    out_shape=[jax.ShapeDtypeStruct((N, DH), jnp.float32)] * 2,
)


def kernel(x, edge_index, W1, b1, W_mu, b_mu, W_ls, b_ls):
    src1 = edge_index[0].astype(jnp.int32)
    dst1 = edge_index[1].astype(jnp.int32)
    dst2 = dst1.reshape(E // K, K)
    z2d = jnp.zeros((K, 128), jnp.float32)
    z1d = jnp.zeros((RPT,), jnp.float32)

    d0, d1 = _deg_kernel(dst2, z1d)
    deg = d0[:N] + d1[:N]
    dinv = jnp.where(deg > 0, lax.rsqrt(deg), 0.0).reshape(N, 1)

    y = _tc1(x, W1, dinv)
    p0, p1 = _pass_kernel(y, src1, dst1, z2d)
    z = _tc2(p0, p1, dinv, b1.reshape(1, -1), W_mu, W_ls)
    q0, q1 = _pass_kernel(z, src1, dst1, z2d)
    mu, ls = _tc3(q0, q1, dinv,
                  b_mu.reshape(1, -1), b_ls.reshape(1, -1))
    return (mu, ls)
